# baseline (speedup 1.0000x reference)
import functools

import jax
import jax.numpy as jnp
from jax import lax
from jax.experimental import pallas as pl
from jax.experimental.pallas import tpu as pltpu

N = 10000
E = 160000
C = 64
NB = 8
P = 5
R_MAX = 10.0
L = 2
G = 8
M = 9
CM = C * M

BE = 512
W = 128
NW = -(-N // W)
NP = NW * W
NUM_CHUNKS = E // BE + NW + 1
E_PAD = NUM_CHUNKS * BE


def _sph_k(u):
    x, y, z = u[:, 0], u[:, 1], u[:, 2]
    s3 = jnp.sqrt(3.0)
    s5 = jnp.sqrt(5.0)
    s15 = jnp.sqrt(15.0)
    return jnp.stack([
        jnp.ones_like(x),
        s3 * x, s3 * y, s3 * z,
        s15 * x * y, s15 * y * z,
        (s5 / 2.0) * (3.0 * z * z - 1.0),
        s15 * x * z, (s15 / 2.0) * (x * x - y * y)
    ], axis=-1)


def _radial_k(r):
    x = r / R_MAX
    n = jnp.arange(1, NB + 1, dtype=jnp.float32)
    bessel = jnp.sqrt(2.0 / R_MAX) * jnp.sin(n[None, :] * jnp.pi * x[:, None]) / jnp.clip(r, 1e-6)[:, None]
    p = float(P)
    env = (1.0 - ((p + 1.0) * (p + 2.0) / 2.0) * x ** P
           + p * (p + 2.0) * x ** (P + 1)
           - (p * (p + 1.0) / 2.0) * x ** (P + 2))
    env = jnp.where(x < 1.0, env, 0.0)
    return bessel * env[:, None]


def _agg_body(nb_ref, sp_ref, ef_ref, sh_ref, loc_ref,
              wr1_ref, br1_ref, wr2_ref, br2_ref, e64_ref, e9_ref,
              out_ref):
    j = pl.program_id(0)

    @pl.when(j == 0)
    def _():
        out_ref[...] = jnp.zeros_like(out_ref)

    hmid = jnp.maximum(
        jnp.dot(ef_ref[...], wr1_ref[...],
                preferred_element_type=jnp.float32) + br1_ref[...], 0.0)
    w = jnp.dot(hmid, wr2_ref[...],
                preferred_element_type=jnp.float32) + br2_ref[...]
    a = w * sp_ref[...]
    msg = (jnp.dot(a, e64_ref[...], preferred_element_type=jnp.float32) *
           jnp.dot(sh_ref[...], e9_ref[...],
                   preferred_element_type=jnp.float32))
    loc = loc_ref[...]
    oh = (lax.broadcasted_iota(jnp.int32, (BE, W), 1) == loc)
    ohf = oh.astype(jnp.float32)
    acc = lax.dot_general(ohf, msg, (((0,), (0,)), ((), ())),
                          preferred_element_type=jnp.float32)
    base = pl.multiple_of(nb_ref[j], W)
    out_ref[pl.ds(base, W), :] += acc


@jax.jit
def _aggregate(node_base, s_p, ef_p, sh_p, loc, Wr1l, br1l, Wr2l, br2l,
               e64, e9):
    return pl.pallas_call(
        _agg_body,
        grid_spec=pltpu.PrefetchScalarGridSpec(
            num_scalar_prefetch=1,
            grid=(NUM_CHUNKS,),
            in_specs=[
                pl.BlockSpec((BE, C), lambda j, nb: (j, 0)),
                pl.BlockSpec((BE, NB), lambda j, nb: (j, 0)),
                pl.BlockSpec((BE, M), lambda j, nb: (j, 0)),
                pl.BlockSpec((BE, 1), lambda j, nb: (j, 0)),
                pl.BlockSpec((NB, C), lambda j, nb: (0, 0)),
                pl.BlockSpec((1, C), lambda j, nb: (0, 0)),
                pl.BlockSpec((C, C), lambda j, nb: (0, 0)),
                pl.BlockSpec((1, C), lambda j, nb: (0, 0)),
                pl.BlockSpec((C, CM), lambda j, nb: (0, 0)),
                pl.BlockSpec((M, CM), lambda j, nb: (0, 0)),
            ],
            out_specs=pl.BlockSpec((NP, CM), lambda j, nb: (0, 0)),
        ),
        out_shape=jax.ShapeDtypeStruct((NP, CM), jnp.float32),
    )(node_base, s_p, ef_p, sh_p, loc, Wr1l, br1l, Wr2l, br2l, e64, e9)


def _node_chain_body(f_ref, sc_ref, alpha_ref, rt_ref, ucat_ref, s9_ref,
                     sum9_ref, out_ref, s_ref):
    bn = f_ref.shape[0]
    f = f_ref[...]
    x = jnp.concatenate([f[:, k * M:(k + 1) * M] for k in range(C)], axis=0)
    fi = jnp.dot(x, rt_ref[:, :81], preferred_element_type=jnp.float32)
    fj = jnp.dot(x, rt_ref[:, 81:], preferred_element_type=jnp.float32)
    outer = fi * fj
    gcat = jnp.dot(outer, ucat_ref[...], preferred_element_type=jnp.float32)
    g2 = gcat[:, :M]
    t = gcat[:, M:]
    g3 = jnp.dot(t * fi, s9_ref[...], preferred_element_type=jnp.float32)
    a0 = alpha_ref[0]
    a1 = alpha_ref[1]
    a2 = alpha_ref[2]
    out = a0 * x + a1 * g2 + a2 * g3
    out_flat = jnp.concatenate(
        [out[k * bn:(k + 1) * bn, :] for k in range(C)], axis=1)
    h_new = out_flat + sc_ref[...]
    out_ref[...] = h_new
    s_ref[...] = jnp.dot(h_new, sum9_ref[...],
                         preferred_element_type=jnp.float32) * (1.0 / M)


@functools.partial(jax.jit, static_argnames=("bn",))
def _node_chain(f, sc, alpha_l, rt, ucat, s9, sum9, bn=200):
    grid = N // bn
    return pl.pallas_call(
        _node_chain_body,
        grid=(grid,),
        in_specs=[
            pl.BlockSpec((bn, CM), lambda i: (i, 0)),
            pl.BlockSpec((bn, CM), lambda i: (i, 0)),
            pl.BlockSpec(memory_space=pltpu.SMEM),
            pl.BlockSpec((M, 162), lambda i: (0, 0)),
            pl.BlockSpec((81, 90), lambda i: (0, 0)),
            pl.BlockSpec((81, M), lambda i: (0, 0)),
            pl.BlockSpec((CM, C), lambda i: (0, 0)),
        ],
        out_specs=[
            pl.BlockSpec((bn, CM), lambda i: (i, 0)),
            pl.BlockSpec((bn, C), lambda i: (i, 0)),
        ],
        out_shape=[
            jax.ShapeDtypeStruct((N, CM), jnp.float32),
            jax.ShapeDtypeStruct((N, C), jnp.float32),
        ],
    )(f, sc, alpha_l, rt, ucat, s9, sum9)


def kernel(atoms, pos, edge_index, batch, emb, Wr1, br1, Wr2, br2, U2, U3,
           alpha, Wp1, bp1, Wp2, bp2):
    src, dst = edge_index[0], edge_index[1]
    vec = pos[src] - pos[dst]
    r = jnp.linalg.norm(vec, axis=-1)
    u = vec / jnp.clip(r, 1e-6)[:, None]
    sh = _sph_k(u)
    ef = _radial_k(r)

    i32 = jnp.int32
    perm = jnp.argsort(dst).astype(i32)
    dst_s = dst[perm].astype(i32)
    wo = jnp.searchsorted(dst_s, (jnp.arange(NW + 1, dtype=i32) * W)).astype(i32)
    k_w = wo[1:] - wo[:-1]
    cw = (k_w + BE - 1) // BE
    cstart = jnp.concatenate([jnp.zeros((1,), i32), jnp.cumsum(cw).astype(i32)])
    cid = jnp.arange(NUM_CHUNKS, dtype=i32)
    w_of_c = jnp.clip(jnp.searchsorted(cstart, cid, side="right") - 1,
                      0, NW - 1).astype(i32)
    p = jnp.arange(E_PAD, dtype=i32)
    cp = p // BE
    wp = w_of_c[cp]
    rank = (cp - cstart[wp]) * BE + p % BE
    valid = rank < k_w[wp]
    spos = jnp.clip(wo[wp] + rank, 0, E - 1)
    eidx = jnp.where(valid, perm[spos], E)
    src_p = jnp.where(valid, src[perm[spos]], 0).astype(i32)
    loc = jnp.where(valid, dst_s[spos] - wp * W, -1).astype(i32)
    loc = loc.reshape(NUM_CHUNKS * BE, 1)
    node_base = (w_of_c * W).astype(i32)

    zrow9 = jnp.zeros((1, M), jnp.float32)
    zrow8 = jnp.zeros((1, NB), jnp.float32)
    sh_p = jnp.concatenate([sh, zrow9], axis=0)[eidx]
    ef_p = jnp.concatenate([ef, zrow8], axis=0)[eidx]

    i9 = jnp.eye(M, dtype=jnp.float32)
    rmat = jnp.kron(i9, jnp.ones((1, M), jnp.float32))
    tmat = jnp.kron(jnp.ones((1, M), jnp.float32), i9)
    rt = jnp.concatenate([rmat, tmat], axis=1)
    s9 = jnp.kron(jnp.ones((M, 1), jnp.float32), i9)
    sum9 = jnp.kron(jnp.eye(C, dtype=jnp.float32),
                    jnp.ones((M, 1), jnp.float32))
    e64 = jnp.kron(jnp.eye(C, dtype=jnp.float32),
                   jnp.ones((1, M), jnp.float32))
    e9 = jnp.kron(jnp.ones((1, C), jnp.float32), i9)

    h = emb[atoms]
    s = h
    for l in range(L):
        s_p = s[src_p]
        f = _aggregate(node_base, s_p, ef_p, sh_p, loc,
                       Wr1[l], br1[l].reshape(1, C), Wr2[l],
                       br2[l].reshape(1, C), e64, e9)
        sc = jnp.pad(h, ((0, 0), (0, CM - h.shape[-1])))
        ucat = jnp.concatenate(
            [U2[l].reshape(81, M), U3[l].reshape(81, 81)], axis=1)
        h, s = _node_chain(f, sc, alpha[l], rt, ucat, s9, sum9)

    hs = h[:, :C]
    pooled = jax.ops.segment_sum(hs, batch, num_segments=G)
    return jax.nn.relu(pooled @ Wp1 + bp1) @ Wp2 + bp2


# speedup vs baseline: 1.2328x; 1.2328x over previous
import functools

import jax
import jax.numpy as jnp
from jax import lax
from jax.experimental import pallas as pl
from jax.experimental.pallas import tpu as pltpu

N = 10000
E = 160000
C = 64
NB = 8
P = 5
R_MAX = 10.0
L = 2
G = 8
M = 9
CM = C * M

BE = 512
W = 128
NW = -(-N // W)
NP = NW * W
NUM_CHUNKS = E // BE + NW + 1
E_PAD = NUM_CHUNKS * BE


def _sph_k(u):
    x, y, z = u[:, 0], u[:, 1], u[:, 2]
    s3 = jnp.sqrt(3.0)
    s5 = jnp.sqrt(5.0)
    s15 = jnp.sqrt(15.0)
    return jnp.stack([
        jnp.ones_like(x),
        s3 * x, s3 * y, s3 * z,
        s15 * x * y, s15 * y * z,
        (s5 / 2.0) * (3.0 * z * z - 1.0),
        s15 * x * z, (s15 / 2.0) * (x * x - y * y)
    ], axis=-1)


def _radial_k(r):
    x = r / R_MAX
    n = jnp.arange(1, NB + 1, dtype=jnp.float32)
    bessel = jnp.sqrt(2.0 / R_MAX) * jnp.sin(n[None, :] * jnp.pi * x[:, None]) / jnp.clip(r, 1e-6)[:, None]
    p = float(P)
    env = (1.0 - ((p + 1.0) * (p + 2.0) / 2.0) * x ** P
           + p * (p + 2.0) * x ** (P + 1)
           - (p * (p + 1.0) / 2.0) * x ** (P + 2))
    env = jnp.where(x < 1.0, env, 0.0)
    return bessel * env[:, None]


def _agg_body(nb_ref, sp_ref, ef_ref, sh_ref, loc_ref,
              wr1_ref, br1_ref, wr2_ref, br2_ref, e64_ref, e9_ref,
              out_ref):
    j = pl.program_id(0)

    @pl.when(j == 0)
    def _():
        out_ref[...] = jnp.zeros_like(out_ref)

    hmid = jnp.maximum(
        jnp.dot(ef_ref[...], wr1_ref[...],
                preferred_element_type=jnp.float32) + br1_ref[...], 0.0)
    w = jnp.dot(hmid, wr2_ref[...],
                preferred_element_type=jnp.float32) + br2_ref[...]
    a = w * sp_ref[...]
    msg = (jnp.dot(a, e64_ref[...], preferred_element_type=jnp.float32) *
           jnp.dot(sh_ref[...], e9_ref[...],
                   preferred_element_type=jnp.float32))
    loc = loc_ref[...]
    oh = (lax.broadcasted_iota(jnp.int32, (BE, W), 1) == loc)
    ohf = oh.astype(jnp.float32)
    acc = lax.dot_general(ohf, msg, (((0,), (0,)), ((), ())),
                          preferred_element_type=jnp.float32)
    base = pl.multiple_of(nb_ref[j], W)
    out_ref[pl.ds(base, W), :] += acc


@jax.jit
def _aggregate(node_base, s_p, ef_p, sh_p, loc, Wr1l, br1l, Wr2l, br2l,
               e64, e9):
    return pl.pallas_call(
        _agg_body,
        grid_spec=pltpu.PrefetchScalarGridSpec(
            num_scalar_prefetch=1,
            grid=(NUM_CHUNKS,),
            in_specs=[
                pl.BlockSpec((BE, C), lambda j, nb: (j, 0)),
                pl.BlockSpec((BE, NB), lambda j, nb: (j, 0)),
                pl.BlockSpec((BE, M), lambda j, nb: (j, 0)),
                pl.BlockSpec((BE, 1), lambda j, nb: (j, 0)),
                pl.BlockSpec((NB, C), lambda j, nb: (0, 0)),
                pl.BlockSpec((1, C), lambda j, nb: (0, 0)),
                pl.BlockSpec((C, C), lambda j, nb: (0, 0)),
                pl.BlockSpec((1, C), lambda j, nb: (0, 0)),
                pl.BlockSpec((C, CM), lambda j, nb: (0, 0)),
                pl.BlockSpec((M, CM), lambda j, nb: (0, 0)),
            ],
            out_specs=pl.BlockSpec((NP, CM), lambda j, nb: (0, 0)),
        ),
        out_shape=jax.ShapeDtypeStruct((NP, CM), jnp.float32),
    )(node_base, s_p, ef_p, sh_p, loc, Wr1l, br1l, Wr2l, br2l, e64, e9)


def _node_chain_body(f_ref, sc_ref, alpha_ref, rt_ref, ucat_ref, s9_ref,
                     sum9_ref, out_ref, s_ref):
    bn = f_ref.shape[0]
    f = f_ref[...]
    x = jnp.concatenate([f[:, k * M:(k + 1) * M] for k in range(C)], axis=0)
    fi = jnp.dot(x, rt_ref[:, :81], preferred_element_type=jnp.float32)
    fj = jnp.dot(x, rt_ref[:, 81:], preferred_element_type=jnp.float32)
    outer = fi * fj
    gcat = jnp.dot(outer, ucat_ref[...], preferred_element_type=jnp.float32)
    g2 = gcat[:, :M]
    t = gcat[:, M:]
    g3 = jnp.dot(t * fi, s9_ref[...], preferred_element_type=jnp.float32)
    a0 = alpha_ref[0]
    a1 = alpha_ref[1]
    a2 = alpha_ref[2]
    out = a0 * x + a1 * g2 + a2 * g3
    out_flat = jnp.concatenate(
        [out[k * bn:(k + 1) * bn, :] for k in range(C)], axis=1)
    h_new = out_flat + sc_ref[...]
    out_ref[...] = h_new
    s_ref[...] = jnp.dot(h_new, sum9_ref[...],
                         preferred_element_type=jnp.float32) * (1.0 / M)


@functools.partial(jax.jit, static_argnames=("bn",))
def _node_chain(f, sc, alpha_l, rt, ucat, s9, sum9, bn=200):
    grid = N // bn
    return pl.pallas_call(
        _node_chain_body,
        grid=(grid,),
        in_specs=[
            pl.BlockSpec((bn, CM), lambda i: (i, 0)),
            pl.BlockSpec((bn, CM), lambda i: (i, 0)),
            pl.BlockSpec(memory_space=pltpu.SMEM),
            pl.BlockSpec((M, 162), lambda i: (0, 0)),
            pl.BlockSpec((81, 90), lambda i: (0, 0)),
            pl.BlockSpec((81, M), lambda i: (0, 0)),
            pl.BlockSpec((CM, C), lambda i: (0, 0)),
        ],
        out_specs=[
            pl.BlockSpec((bn, CM), lambda i: (i, 0)),
            pl.BlockSpec((bn, C), lambda i: (i, 0)),
        ],
        out_shape=[
            jax.ShapeDtypeStruct((N, CM), jnp.float32),
            jax.ShapeDtypeStruct((N, C), jnp.float32),
        ],
    )(f, sc, alpha_l, rt, ucat, s9, sum9)


def kernel(atoms, pos, edge_index, batch, emb, Wr1, br1, Wr2, br2, U2, U3,
           alpha, Wp1, bp1, Wp2, bp2):
    src, dst = edge_index[0], edge_index[1]
    vec = pos[src] - pos[dst]
    r = jnp.linalg.norm(vec, axis=-1)
    u = vec / jnp.clip(r, 1e-6)[:, None]
    sh = _sph_k(u)
    ef = _radial_k(r)

    i32 = jnp.int32
    perm = jnp.argsort(dst).astype(i32)
    dst_s = dst[perm].astype(i32)
    wo = jnp.searchsorted(dst_s, (jnp.arange(NW + 1, dtype=i32) * W)).astype(i32)
    k_w = wo[1:] - wo[:-1]
    cw = (k_w + BE - 1) // BE
    cstart = jnp.concatenate([jnp.zeros((1,), i32), jnp.cumsum(cw).astype(i32)])
    cid = jnp.arange(NUM_CHUNKS, dtype=i32)
    w_of_c = jnp.clip(jnp.searchsorted(cstart, cid, side="right") - 1,
                      0, NW - 1).astype(i32)
    p = jnp.arange(E_PAD, dtype=i32)
    cp = p // BE
    wp = w_of_c[cp]
    rank = (cp - cstart[wp]) * BE + p % BE
    valid = rank < k_w[wp]
    spos = jnp.clip(wo[wp] + rank, 0, E - 1)
    pad_e = E + (p % 1024)
    pad_n = p % N
    eidx = jnp.where(valid, perm[spos], pad_e)
    src_p = jnp.where(valid, src[perm[spos]], pad_n).astype(i32)
    loc = jnp.where(valid, dst_s[spos] - wp * W, -1).astype(i32)
    loc = loc.reshape(NUM_CHUNKS * BE, 1)
    node_base = (w_of_c * W).astype(i32)

    zrow9 = jnp.zeros((1024, M), jnp.float32)
    zrow8 = jnp.zeros((1024, NB), jnp.float32)
    sh_p = jnp.concatenate([sh, zrow9], axis=0)[eidx]
    ef_p = jnp.concatenate([ef, zrow8], axis=0)[eidx]

    i9 = jnp.eye(M, dtype=jnp.float32)
    rmat = jnp.kron(i9, jnp.ones((1, M), jnp.float32))
    tmat = jnp.kron(jnp.ones((1, M), jnp.float32), i9)
    rt = jnp.concatenate([rmat, tmat], axis=1)
    s9 = jnp.kron(jnp.ones((M, 1), jnp.float32), i9)
    sum9 = jnp.kron(jnp.eye(C, dtype=jnp.float32),
                    jnp.ones((M, 1), jnp.float32))
    e64 = jnp.kron(jnp.eye(C, dtype=jnp.float32),
                   jnp.ones((1, M), jnp.float32))
    e9 = jnp.kron(jnp.ones((1, C), jnp.float32), i9)

    h = emb[atoms]
    s = h
    for l in range(L):
        s_p = s[src_p]
        f = _aggregate(node_base, s_p, ef_p, sh_p, loc,
                       Wr1[l], br1[l].reshape(1, C), Wr2[l],
                       br2[l].reshape(1, C), e64, e9)
        sc = jnp.pad(h, ((0, 0), (0, CM - h.shape[-1])))
        ucat = jnp.concatenate(
            [U2[l].reshape(81, M), U3[l].reshape(81, 81)], axis=1)
        h, s = _node_chain(f, sc, alpha[l], rt, ucat, s9, sum9)

    hs = h[:, :C]
    pooled = jax.ops.segment_sum(hs, batch, num_segments=G)
    return jax.nn.relu(pooled @ Wp1 + bp1) @ Wp2 + bp2


# speedup vs baseline: 2.7820x; 2.2567x over previous
import functools

import jax
import jax.numpy as jnp
from jax import lax
from jax.experimental import pallas as pl
from jax.experimental.pallas import tpu as pltpu

N = 10000
E = 160000
C = 64
NB = 8
P = 5
R_MAX = 10.0
L = 2
G = 8
M = 9
CM = C * M

BE = 512
W = 128
NW = -(-N // W)
NP = NW * W
NUM_CHUNKS = E // BE + NW + 1
E_PAD = NUM_CHUNKS * BE
TC = 32
NZPAD = 1024


def _sph_k(u):
    x, y, z = u[:, 0], u[:, 1], u[:, 2]
    s3 = jnp.sqrt(3.0)
    s5 = jnp.sqrt(5.0)
    s15 = jnp.sqrt(15.0)
    return jnp.stack([
        jnp.ones_like(x),
        s3 * x, s3 * y, s3 * z,
        s15 * x * y, s15 * y * z,
        (s5 / 2.0) * (3.0 * z * z - 1.0),
        s15 * x * z, (s15 / 2.0) * (x * x - y * y)
    ], axis=-1)


def _radial_k(r):
    x = r / R_MAX
    n = jnp.arange(1, NB + 1, dtype=jnp.float32)
    bessel = jnp.sqrt(2.0 / R_MAX) * jnp.sin(n[None, :] * jnp.pi * x[:, None]) / jnp.clip(r, 1e-6)[:, None]
    p = float(P)
    env = (1.0 - ((p + 1.0) * (p + 2.0) / 2.0) * x ** P
           + p * (p + 2.0) * x ** (P + 1)
           - (p * (p + 1.0) / 2.0) * x ** (P + 2))
    env = jnp.where(x < 1.0, env, 0.0)
    return bessel * env[:, None]


def _agg_body(nb_ref, tab_ref, sp_ref,
              wr1_ref, br1_ref, wr2_ref, br2_ref, e64_ref, e9_ref,
              out_ref):
    j = pl.program_id(0)

    @pl.when(j == 0)
    def _():
        out_ref[...] = jnp.zeros_like(out_ref)

    tab = tab_ref[...]
    sh = tab[:, 2:2 + M]
    ef = tab[:, 11:11 + NB]
    hmid = jnp.maximum(
        jnp.dot(ef, wr1_ref[...],
                preferred_element_type=jnp.float32) + br1_ref[...], 0.0)
    w = jnp.dot(hmid, wr2_ref[...],
                preferred_element_type=jnp.float32) + br2_ref[...]
    a = w * sp_ref[...]
    msg = (jnp.dot(a, e64_ref[...], preferred_element_type=jnp.float32) *
           jnp.dot(sh, e9_ref[...],
                   preferred_element_type=jnp.float32))
    base = pl.multiple_of(nb_ref[j], W)
    loc = tab[:, 0:1].astype(jnp.int32) - base
    oh = (lax.broadcasted_iota(jnp.int32, (BE, W), 1) == loc)
    ohf = oh.astype(jnp.float32)
    acc = lax.dot_general(ohf, msg, (((0,), (0,)), ((), ())),
                          preferred_element_type=jnp.float32)
    out_ref[pl.ds(base, W), :] += acc


@jax.jit
def _aggregate(node_base, tab_p, s_p, Wr1l, br1l, Wr2l, br2l, e64, e9):
    return pl.pallas_call(
        _agg_body,
        grid_spec=pltpu.PrefetchScalarGridSpec(
            num_scalar_prefetch=1,
            grid=(NUM_CHUNKS,),
            in_specs=[
                pl.BlockSpec((BE, TC), lambda j, nb: (j, 0)),
                pl.BlockSpec((BE, C), lambda j, nb: (j, 0)),
                pl.BlockSpec((NB, C), lambda j, nb: (0, 0)),
                pl.BlockSpec((1, C), lambda j, nb: (0, 0)),
                pl.BlockSpec((C, C), lambda j, nb: (0, 0)),
                pl.BlockSpec((1, C), lambda j, nb: (0, 0)),
                pl.BlockSpec((C, CM), lambda j, nb: (0, 0)),
                pl.BlockSpec((M, CM), lambda j, nb: (0, 0)),
            ],
            out_specs=pl.BlockSpec((NP, CM), lambda j, nb: (0, 0)),
        ),
        out_shape=jax.ShapeDtypeStruct((NP, CM), jnp.float32),
    )(node_base, tab_p, s_p, Wr1l, br1l, Wr2l, br2l, e64, e9)


def _node_chain_body(f_ref, sc_ref, alpha_ref, rt_ref, ucat_ref, s9_ref,
                     sum9_ref, out_ref, s_ref):
    bn = f_ref.shape[0]
    f = f_ref[...]
    x = jnp.concatenate([f[:, k * M:(k + 1) * M] for k in range(C)], axis=0)
    fi = jnp.dot(x, rt_ref[:, :81], preferred_element_type=jnp.float32)
    fj = jnp.dot(x, rt_ref[:, 81:], preferred_element_type=jnp.float32)
    outer = fi * fj
    gcat = jnp.dot(outer, ucat_ref[...], preferred_element_type=jnp.float32)
    g2 = gcat[:, :M]
    t = gcat[:, M:]
    g3 = jnp.dot(t * fi, s9_ref[...], preferred_element_type=jnp.float32)
    a0 = alpha_ref[0]
    a1 = alpha_ref[1]
    a2 = alpha_ref[2]
    out = a0 * x + a1 * g2 + a2 * g3
    out_flat = jnp.concatenate(
        [out[k * bn:(k + 1) * bn, :] for k in range(C)], axis=1)
    h_new = out_flat + sc_ref[...]
    out_ref[...] = h_new
    s_ref[...] = jnp.dot(h_new, sum9_ref[...],
                         preferred_element_type=jnp.float32) * (1.0 / M)


@functools.partial(jax.jit, static_argnames=("bn",))
def _node_chain(f, sc, alpha_l, rt, ucat, s9, sum9, bn=200):
    grid = N // bn
    return pl.pallas_call(
        _node_chain_body,
        grid=(grid,),
        in_specs=[
            pl.BlockSpec((bn, CM), lambda i: (i, 0)),
            pl.BlockSpec((bn, CM), lambda i: (i, 0)),
            pl.BlockSpec(memory_space=pltpu.SMEM),
            pl.BlockSpec((M, 162), lambda i: (0, 0)),
            pl.BlockSpec((81, 90), lambda i: (0, 0)),
            pl.BlockSpec((81, M), lambda i: (0, 0)),
            pl.BlockSpec((CM, C), lambda i: (0, 0)),
        ],
        out_specs=[
            pl.BlockSpec((bn, CM), lambda i: (i, 0)),
            pl.BlockSpec((bn, C), lambda i: (i, 0)),
        ],
        out_shape=[
            jax.ShapeDtypeStruct((N, CM), jnp.float32),
            jax.ShapeDtypeStruct((N, C), jnp.float32),
        ],
    )(f, sc, alpha_l, rt, ucat, s9, sum9)


def kernel(atoms, pos, edge_index, batch, emb, Wr1, br1, Wr2, br2, U2, U3,
           alpha, Wp1, bp1, Wp2, bp2):
    src, dst = edge_index[0], edge_index[1]
    vec = pos[src] - pos[dst]
    r = jnp.linalg.norm(vec, axis=-1)
    u = vec / jnp.clip(r, 1e-6)[:, None]
    sh = _sph_k(u)
    ef = _radial_k(r)

    i32 = jnp.int32
    table = jnp.concatenate([
        dst.astype(jnp.float32)[:, None],
        src.astype(jnp.float32)[:, None],
        sh, ef,
        jnp.zeros((E, TC - 19), jnp.float32),
    ], axis=1)

    perm = jnp.argsort(dst).astype(i32)
    table_s = table[perm]
    dst_s = table_s[:, 0].astype(i32)
    wo = jnp.searchsorted(dst_s, (jnp.arange(NW + 1, dtype=i32) * W)).astype(i32)
    k_w = wo[1:] - wo[:-1]
    cw = (k_w + BE - 1) // BE
    cstart = jnp.concatenate([jnp.zeros((1,), i32), jnp.cumsum(cw).astype(i32)])
    cid = jnp.arange(NUM_CHUNKS, dtype=i32)
    w_of_c = jnp.clip(jnp.searchsorted(cstart, cid, side="right") - 1,
                      0, NW - 1).astype(i32)
    rank0_c = (cid - cstart[w_of_c]) * BE
    kw_c = k_w[w_of_c]
    wo_c = wo[w_of_c]
    pmod = jnp.tile(jnp.arange(BE, dtype=i32), NUM_CHUNKS)
    rank = jnp.repeat(rank0_c, BE) + pmod
    valid = rank < jnp.repeat(kw_c, BE)
    spos = jnp.repeat(wo_c, BE) + rank
    p = jnp.arange(E_PAD, dtype=i32)
    spos2 = jnp.where(valid, jnp.clip(spos, 0, E - 1), E + p % NZPAD)
    zpad = jnp.zeros((NZPAD, TC), jnp.float32)
    zpad = zpad.at[:, 1].set((jnp.arange(NZPAD) * 97 % N).astype(jnp.float32))
    tab_p = jnp.concatenate([table_s, zpad], axis=0)[spos2]
    src_p = tab_p[:, 1].astype(i32)
    node_base = (w_of_c * W).astype(i32)

    i9 = jnp.eye(M, dtype=jnp.float32)
    rmat = jnp.kron(i9, jnp.ones((1, M), jnp.float32))
    tmat = jnp.kron(jnp.ones((1, M), jnp.float32), i9)
    rt = jnp.concatenate([rmat, tmat], axis=1)
    s9 = jnp.kron(jnp.ones((M, 1), jnp.float32), i9)
    sum9 = jnp.kron(jnp.eye(C, dtype=jnp.float32),
                    jnp.ones((M, 1), jnp.float32))
    e64 = jnp.kron(jnp.eye(C, dtype=jnp.float32),
                   jnp.ones((1, M), jnp.float32))
    e9 = jnp.kron(jnp.ones((1, C), jnp.float32), i9)

    h = emb[atoms]
    s = h
    for l in range(L):
        s_p = s[src_p]
        f = _aggregate(node_base, tab_p, s_p,
                       Wr1[l], br1[l].reshape(1, C), Wr2[l],
                       br2[l].reshape(1, C), e64, e9)
        sc = jnp.pad(h, ((0, 0), (0, CM - h.shape[-1])))
        ucat = jnp.concatenate(
            [U2[l].reshape(81, M), U3[l].reshape(81, 81)], axis=1)
        h, s = _node_chain(f, sc, alpha[l], rt, ucat, s9, sum9)

    hs = h[:, :C]
    pooled = jax.ops.segment_sum(hs, batch, num_segments=G)
    return jax.nn.relu(pooled @ Wp1 + bp1) @ Wp2 + bp2
